# Initial kernel scaffold; baseline (speedup 1.0000x reference)
import jax
import jax.numpy as jnp
from jax.experimental import pallas as pl


def kernel(y_pred, y_true):
    raise NotImplementedError("write your pallas kernel here")


# speedup vs baseline: 1.1591x; 1.1591x over previous
import functools

import jax
import jax.numpy as jnp
from jax.experimental import pallas as pl
from jax.experimental.pallas import tpu as pltpu

_BETA2 = 1.5 * 1.5
_COEF = 1.0 + _BETA2


def _tc_body(x_ref, t_ref, out_ref, tp_acc, s_acc, *, hw):
    b = pl.program_id(0)
    h = pl.program_id(1)
    nb = pl.num_programs(0)
    nh = pl.num_programs(1)

    x = x_ref[0]
    t = t_ref[0]
    p = 1.0 / (1.0 + jnp.exp(-x))
    cls = jax.lax.broadcasted_iota(jnp.int32, x.shape, 0)
    sel = jnp.where(cls == t[None], p, 0.0)
    s_part = jnp.sum(p, axis=(0, 1))
    tp_part = jnp.sum(sel, axis=(0, 1))

    @pl.when((b == 0) & (h == 0))
    def _():
        tp_acc[...] = jnp.zeros_like(tp_acc)
        s_acc[...] = jnp.zeros_like(s_acc)

    row = jax.lax.broadcasted_iota(jnp.int32, tp_acc.shape, 0)
    hit = row == b
    tp_acc[...] += jnp.where(hit, tp_part[None, :], 0.0)
    s_acc[...] += jnp.where(hit, s_part[None, :], 0.0)

    @pl.when((b == nb - 1) & (h == nh - 1))
    def _():
        tp = jnp.sum(tp_acc[...], axis=1)
        s = jnp.sum(s_acc[...], axis=1)
        f = _COEF * tp / (s + _BETA2 * hw)
        out_ref[...] = jnp.mean(1.0 - f)[None, None]


def kernel(y_pred, y_true):
    B, C, H, W = y_pred.shape
    HB = 64
    nh = H // HB
    out = pl.pallas_call(
        functools.partial(_tc_body, hw=float(H * W)),
        grid=(B, nh),
        in_specs=[
            pl.BlockSpec((1, C, HB, W), lambda b, h: (b, 0, h, 0)),
            pl.BlockSpec((1, HB, W), lambda b, h: (b, h, 0)),
        ],
        out_specs=pl.BlockSpec((1, 1), lambda b, h: (0, 0)),
        out_shape=jax.ShapeDtypeStruct((1, 1), jnp.float32),
        scratch_shapes=[
            pltpu.VMEM((B, W), jnp.float32),
            pltpu.VMEM((B, W), jnp.float32),
        ],
    )(y_pred, y_true)
    return out[0, 0]


# speedup vs baseline: 1.3452x; 1.1605x over previous
import functools

import jax
import jax.numpy as jnp
from jax.experimental import pallas as pl
from jax.experimental.pallas import tpu as pltpu

_BETA2 = 1.5 * 1.5
_COEF = 1.0 + _BETA2


def _tc_body(x_ref, t_ref, out_ref, tp_acc, s_acc, *, hw):
    b = pl.program_id(0)
    h = pl.program_id(1)
    nb = pl.num_programs(0)
    nh = pl.num_programs(1)

    x = x_ref[0]
    t = t_ref[0]
    th = jnp.tanh(x * 0.5)
    cls = jax.lax.broadcasted_iota(jnp.int32, x.shape, 0)
    s_part = jnp.sum(th, axis=(0, 1))
    tp_part = jnp.sum(jnp.where(cls == t[None], th, 0.0), axis=(0, 1))

    @pl.when((b == 0) & (h == 0))
    def _():
        tp_acc[...] = jnp.zeros_like(tp_acc)
        s_acc[...] = jnp.zeros_like(s_acc)

    row = jax.lax.broadcasted_iota(jnp.int32, tp_acc.shape, 0)
    hit = row == b
    tp_acc[...] += jnp.where(hit, tp_part[None, :], 0.0)
    s_acc[...] += jnp.where(hit, s_part[None, :], 0.0)

    @pl.when((b == nb - 1) & (h == nh - 1))
    def _():
        c = x_ref.shape[1]
        tp = 0.5 * hw + 0.5 * jnp.sum(tp_acc[...], axis=1)
        s = 0.5 * (c * hw) + 0.5 * jnp.sum(s_acc[...], axis=1)
        f = _COEF * tp / (s + _BETA2 * hw)
        out_ref[...] = jnp.mean(1.0 - f)[None, None]


def kernel(y_pred, y_true):
    B, C, H, W = y_pred.shape
    HB = 64
    nh = H // HB
    out = pl.pallas_call(
        functools.partial(_tc_body, hw=float(H * W)),
        grid=(B, nh),
        in_specs=[
            pl.BlockSpec((1, C, HB, W), lambda b, h: (b, 0, h, 0)),
            pl.BlockSpec((1, HB, W), lambda b, h: (b, h, 0)),
        ],
        out_specs=pl.BlockSpec((1, 1), lambda b, h: (0, 0)),
        out_shape=jax.ShapeDtypeStruct((1, 1), jnp.float32),
        scratch_shapes=[
            pltpu.VMEM((B, W), jnp.float32),
            pltpu.VMEM((B, W), jnp.float32),
        ],
    )(y_pred, y_true)
    return out[0, 0]


# speedup vs baseline: 1.6921x; 1.2579x over previous
import functools

import jax
import jax.numpy as jnp
from jax.experimental import pallas as pl
from jax.experimental.pallas import tpu as pltpu

_BETA2 = 1.5 * 1.5
_COEF = 1.0 + _BETA2


def _tc_body(x_ref, t_ref, out_ref, tp_acc, s_acc, *, hw):
    b = pl.program_id(0)
    h = pl.program_id(1)
    nb = pl.num_programs(0)
    nh = pl.num_programs(1)

    x = x_ref[0]
    t = t_ref[0]
    th = jnp.tanh(x * 0.5)
    cls = jax.lax.broadcasted_iota(jnp.int32, x.shape, 0)
    s_part = jnp.sum(th, axis=(0, 1))
    tp_part = jnp.sum(jnp.where(cls == t[None], th, 0.0), axis=(0, 1))

    @pl.when((b == 0) & (h == 0))
    def _():
        tp_acc[...] = jnp.zeros_like(tp_acc)
        s_acc[...] = jnp.zeros_like(s_acc)

    row = jax.lax.broadcasted_iota(jnp.int32, tp_acc.shape, 0)
    hit = row == b
    tp_acc[...] += jnp.where(hit, tp_part[None, :], 0.0)
    s_acc[...] += jnp.where(hit, s_part[None, :], 0.0)

    @pl.when((b == nb - 1) & (h == nh - 1))
    def _():
        c = x_ref.shape[1]
        tp = 0.5 * hw + 0.5 * jnp.sum(tp_acc[...], axis=1)
        s = 0.5 * (c * hw) + 0.5 * jnp.sum(s_acc[...], axis=1)
        f = _COEF * tp / (s + _BETA2 * hw)
        out_ref[...] = jnp.mean(1.0 - f)[None, None]


def kernel(y_pred, y_true):
    B, C, H, W = y_pred.shape
    HB = 128
    nh = H // HB
    out = pl.pallas_call(
        functools.partial(_tc_body, hw=float(H * W)),
        grid=(B, nh),
        in_specs=[
            pl.BlockSpec((1, C, HB, W), lambda b, h: (b, 0, h, 0)),
            pl.BlockSpec((1, HB, W), lambda b, h: (b, h, 0)),
        ],
        out_specs=pl.BlockSpec((1, 1), lambda b, h: (0, 0)),
        out_shape=jax.ShapeDtypeStruct((1, 1), jnp.float32),
        scratch_shapes=[
            pltpu.VMEM((B, W), jnp.float32),
            pltpu.VMEM((B, W), jnp.float32),
        ],
    )(y_pred, y_true)
    return out[0, 0]


# speedup vs baseline: 1.8911x; 1.1176x over previous
import functools

import jax
import jax.numpy as jnp
from jax.experimental import pallas as pl
from jax.experimental.pallas import tpu as pltpu

_BETA2 = 1.5 * 1.5
_COEF = 1.0 + _BETA2


def _tc_body(x_ref, t_ref, out_ref, tp_acc, s_acc, *, hw):
    b = pl.program_id(0)
    h = pl.program_id(1)
    nb = pl.num_programs(0)
    nh = pl.num_programs(1)

    x = x_ref[0]
    t = t_ref[0]
    th = jnp.tanh(x * 0.5)
    cls = jax.lax.broadcasted_iota(jnp.int32, x.shape, 0)
    s_part = jnp.sum(th, axis=(0, 1))
    tp_part = jnp.sum(jnp.where(cls == t[None], th, 0.0), axis=(0, 1))

    @pl.when((b == 0) & (h == 0))
    def _():
        tp_acc[...] = jnp.zeros_like(tp_acc)
        s_acc[...] = jnp.zeros_like(s_acc)

    row = jax.lax.broadcasted_iota(jnp.int32, tp_acc.shape, 0)
    hit = row == b
    tp_acc[...] += jnp.where(hit, tp_part[None, :], 0.0)
    s_acc[...] += jnp.where(hit, s_part[None, :], 0.0)

    @pl.when((b == nb - 1) & (h == nh - 1))
    def _():
        c = x_ref.shape[1]
        tp = 0.5 * hw + 0.5 * jnp.sum(tp_acc[...], axis=1)
        s = 0.5 * (c * hw) + 0.5 * jnp.sum(s_acc[...], axis=1)
        f = _COEF * tp / (s + _BETA2 * hw)
        out_ref[...] = jnp.mean(1.0 - f)[None, None]


def kernel(y_pred, y_true):
    B, C, H, W = y_pred.shape
    HB = 256
    nh = H // HB
    out = pl.pallas_call(
        functools.partial(_tc_body, hw=float(H * W)),
        grid=(B, nh),
        in_specs=[
            pl.BlockSpec((1, C, HB, W), lambda b, h: (b, 0, h, 0)),
            pl.BlockSpec((1, HB, W), lambda b, h: (b, h, 0)),
        ],
        out_specs=pl.BlockSpec((1, 1), lambda b, h: (0, 0)),
        out_shape=jax.ShapeDtypeStruct((1, 1), jnp.float32),
        scratch_shapes=[
            pltpu.VMEM((B, W), jnp.float32),
            pltpu.VMEM((B, W), jnp.float32),
        ],
    )(y_pred, y_true)
    return out[0, 0]
